# baseline (speedup 1.0000x reference)
import functools

import jax
import jax.numpy as jnp
from jax import lax
from jax.experimental import pallas as pl
from jax.experimental.pallas import tpu as pltpu
from jax.experimental.pallas import tpu_sc as plsc

D = 64
DP = 128
BATCH = 16384

NC = 1
NS = 16
L = 16
NW = NC * NS
BPW = BATCH // NW
CHUNK = 128
NCHUNK = BPW // CHUNK
NBUF = 2


def _mf_body(user_h, pos_h, neg_h, umat_h, imat_h, out_h,
             idx_u, idx_p, idx_n, pid_u, pid_p, pid_n,
             rows_u, rows_p, rows_n, out_v, sem0, sem1):
    c = lax.axis_index("c")
    s = lax.axis_index("s")
    wid = s * NC + c
    base = wid * BPW
    sems = (sem0, sem1)

    for k in range(NCHUNK):
        off = base + k * CHUNK
        pltpu.sync_copy(user_h.at[pl.ds(off, CHUNK)], idx_u.at[k])
        pltpu.sync_copy(pos_h.at[pl.ds(off, CHUNK)], idx_p.at[k])
        pltpu.sync_copy(neg_h.at[pl.ds(off, CHUNK)], idx_n.at[k])
    for k in range(NCHUNK):
        for v in range(CHUNK // L):
            sl = pl.ds(v * L, L)
            pid_u[k, sl] = idx_u[k, sl] >> 1
            pid_p[k, sl] = idx_p[k, sl] >> 1
            pid_n[k, sl] = idx_n[k, sl] >> 1

    iot = lax.iota(jnp.int32, L)

    def fire(k):
        b = k % NBUF
        sem = sems[b]
        return (
            pltpu.async_copy(umat_h.at[pid_u.at[k]], rows_u.at[b], sem),
            pltpu.async_copy(imat_h.at[pid_p.at[k]], rows_p.at[b], sem),
            pltpu.async_copy(imat_h.at[pid_n.at[k]], rows_n.at[b], sem),
        )

    def compute(k):
        b = k % NBUF
        ru, rp, rn = rows_u.at[b], rows_p.at[b], rows_n.at[b]

        def group_body(g, carry):
            rbase = g * L
            rows16 = rbase + iot
            sl = pl.ds(rbase, L)
            hu = (idx_u[k, sl] & 1) * D
            hp = (idx_p[k, sl] & 1) * D
            hn = (idx_n[k, sl] & 1) * D
            acc = jnp.zeros((L,), jnp.float32)
            for d in range(D):
                uv = plsc.load_gather(ru, [rows16, hu + d])
                iv = plsc.load_gather(rp, [rows16, hp + d])
                jv = plsc.load_gather(rn, [rows16, hn + d])
                acc = acc + uv * (iv - jv)
            out_v[pl.ds(k * CHUNK + rbase, L)] = acc
            return carry

        lax.fori_loop(0, CHUNK // L, group_body, 0)

    pending = {}
    for k in range(NCHUNK + 1):
        if k < NCHUNK:
            pending[k] = fire(k)
        if k >= 1:
            for cp in pending.pop(k - 1):
                cp.wait()
            compute(k - 1)

    pltpu.sync_copy(out_v, out_h.at[pl.ds(base, BPW)])


@jax.jit
def _mf(user, pos, neg, umat2, imat2):
    mesh = plsc.VectorSubcoreMesh(core_axis_name="c", subcore_axis_name="s", num_cores=1)
    kfn = functools.partial(
        pl.kernel,
        out_type=jax.ShapeDtypeStruct((BATCH,), jnp.float32),
        mesh=mesh,
        compiler_params=pltpu.CompilerParams(
            needs_layout_passes=False, use_tc_tiling_on_sc=True),
        scratch_types=[
            pltpu.VMEM((NCHUNK, CHUNK), jnp.int32),
            pltpu.VMEM((NCHUNK, CHUNK), jnp.int32),
            pltpu.VMEM((NCHUNK, CHUNK), jnp.int32),
            pltpu.VMEM((NCHUNK, CHUNK), jnp.int32),
            pltpu.VMEM((NCHUNK, CHUNK), jnp.int32),
            pltpu.VMEM((NCHUNK, CHUNK), jnp.int32),
            pltpu.VMEM((NBUF, CHUNK, DP), jnp.float32),
            pltpu.VMEM((NBUF, CHUNK, DP), jnp.float32),
            pltpu.VMEM((NBUF, CHUNK, DP), jnp.float32),
            pltpu.VMEM((BPW,), jnp.float32),
            pltpu.SemaphoreType.DMA,
            pltpu.SemaphoreType.DMA,
        ],
    )(_mf_body)
    return kfn(user, pos, neg, umat2, imat2)


def kernel(user, pos, neg, user_mat, item_mat):
    user = user.astype(jnp.int32)
    pos = pos.astype(jnp.int32)
    neg = neg.astype(jnp.int32)
    umat2 = user_mat.reshape(user_mat.shape[0] // 2, DP)
    imat2 = item_mat.reshape(item_mat.shape[0] // 2, DP)
    return _mf(user, pos, neg, umat2, imat2)


# speedup vs baseline: 1.0437x; 1.0437x over previous
import functools

import jax
import jax.numpy as jnp
from jax import lax
from jax.experimental import pallas as pl
from jax.experimental.pallas import tpu as pltpu
from jax.experimental.pallas import tpu_sc as plsc

D = 64
DP = 128
BATCH = 16384

NC = 2
NS = 16
L = 16
NW = NC * NS
BPW = BATCH // NW
CHUNK = 128
NCHUNK = BPW // CHUNK
NBUF = 2


def _mf_body(user_h, pos_h, neg_h, umat_h, imat_h, out_h,
             idx_u, idx_p, idx_n, pid_u, pid_p, pid_n,
             rows_u, rows_p, rows_n, out_v, sem0, sem1):
    c = lax.axis_index("c")
    s = lax.axis_index("s")
    wid = s * NC + c
    base = wid * BPW
    sems = (sem0, sem1)

    for k in range(NCHUNK):
        off = base + k * CHUNK
        pltpu.sync_copy(user_h.at[pl.ds(off, CHUNK)], idx_u.at[k])
        pltpu.sync_copy(pos_h.at[pl.ds(off, CHUNK)], idx_p.at[k])
        pltpu.sync_copy(neg_h.at[pl.ds(off, CHUNK)], idx_n.at[k])
    for k in range(NCHUNK):
        for v in range(CHUNK // L):
            sl = pl.ds(v * L, L)
            pid_u[k, sl] = idx_u[k, sl] >> 1
            pid_p[k, sl] = idx_p[k, sl] >> 1
            pid_n[k, sl] = idx_n[k, sl] >> 1

    iot = lax.iota(jnp.int32, L)

    def fire(k):
        b = k % NBUF
        sem = sems[b]
        return (
            pltpu.async_copy(umat_h.at[pid_u.at[k]], rows_u.at[b], sem),
            pltpu.async_copy(imat_h.at[pid_p.at[k]], rows_p.at[b], sem),
            pltpu.async_copy(imat_h.at[pid_n.at[k]], rows_n.at[b], sem),
        )

    def compute(k):
        b = k % NBUF
        ru, rp, rn = rows_u.at[b], rows_p.at[b], rows_n.at[b]

        def group_body(g, carry):
            rbase = g * L
            rows16 = rbase + iot
            sl = pl.ds(rbase, L)
            hu = (idx_u[k, sl] & 1) * D
            hp = (idx_p[k, sl] & 1) * D
            hn = (idx_n[k, sl] & 1) * D
            acc = jnp.zeros((L,), jnp.float32)
            for d in range(D):
                uv = plsc.load_gather(ru, [rows16, hu + d])
                iv = plsc.load_gather(rp, [rows16, hp + d])
                jv = plsc.load_gather(rn, [rows16, hn + d])
                acc = acc + uv * (iv - jv)
            out_v[pl.ds(k * CHUNK + rbase, L)] = acc
            return carry

        lax.fori_loop(0, CHUNK // L, group_body, 0)

    pending = {}
    for k in range(NCHUNK + 1):
        if k < NCHUNK:
            pending[k] = fire(k)
        if k >= 1:
            for cp in pending.pop(k - 1):
                cp.wait()
            compute(k - 1)

    pltpu.sync_copy(out_v, out_h.at[pl.ds(base, BPW)])


@jax.jit
def _mf(user, pos, neg, umat2, imat2):
    mesh = plsc.VectorSubcoreMesh(core_axis_name="c", subcore_axis_name="s")
    kfn = functools.partial(
        pl.kernel,
        out_type=jax.ShapeDtypeStruct((BATCH,), jnp.float32),
        mesh=mesh,
        compiler_params=pltpu.CompilerParams(
            needs_layout_passes=False, use_tc_tiling_on_sc=True),
        scratch_types=[
            pltpu.VMEM((NCHUNK, CHUNK), jnp.int32),
            pltpu.VMEM((NCHUNK, CHUNK), jnp.int32),
            pltpu.VMEM((NCHUNK, CHUNK), jnp.int32),
            pltpu.VMEM((NCHUNK, CHUNK), jnp.int32),
            pltpu.VMEM((NCHUNK, CHUNK), jnp.int32),
            pltpu.VMEM((NCHUNK, CHUNK), jnp.int32),
            pltpu.VMEM((NBUF, CHUNK, DP), jnp.float32),
            pltpu.VMEM((NBUF, CHUNK, DP), jnp.float32),
            pltpu.VMEM((NBUF, CHUNK, DP), jnp.float32),
            pltpu.VMEM((BPW,), jnp.float32),
            pltpu.SemaphoreType.DMA,
            pltpu.SemaphoreType.DMA,
        ],
    )(_mf_body)
    return kfn(user, pos, neg, umat2, imat2)


def kernel(user, pos, neg, user_mat, item_mat):
    user = user.astype(jnp.int32)
    pos = pos.astype(jnp.int32)
    neg = neg.astype(jnp.int32)
    umat2 = user_mat.reshape(user_mat.shape[0] // 2, DP)
    imat2 = item_mat.reshape(item_mat.shape[0] // 2, DP)
    return _mf(user, pos, neg, umat2, imat2)


# speedup vs baseline: 1.3205x; 1.2653x over previous
import functools

import jax
import jax.numpy as jnp
from jax import lax
from jax.experimental import pallas as pl
from jax.experimental.pallas import tpu as pltpu
from jax.experimental.pallas import tpu_sc as plsc

D = 64
DP = 128
BATCH = 16384

NC = 2
NS = 16
L = 16
NW = NC * NS
BPW = BATCH // NW
CHUNK = 128
NCHUNK = BPW // CHUNK
WAVE = 4
NSLOT = 8
WPC = CHUNK // WAVE
GPC = CHUNK // L

def _lane_sum(v):
    iota = lax.iota(jnp.int32, 16)
    for s in (8, 4, 2, 1):
        v = v + jnp.take_along_axis(v, iota ^ s, axis=0)
    return v


def _mf_body(user_h, pos_h, neg_h, umat_t, imat_h, out_h,
             idx_u, idx_p, idx_n, pid_p, pid_n,
             rows_p, rows_n, diff, ublk, out_v,
             sem_i, sem_a, sem_b):
    c = lax.axis_index("c")
    s = lax.axis_index("s")
    wid = s * NC + c
    base = wid * BPW
    sems = (sem_a, sem_b)
    iot = lax.iota(jnp.int32, L)

    pltpu.sync_copy(user_h.at[pl.ds(base, BPW)], idx_u.at[pl.ds(0, BPW)])
    for k in range(NCHUNK):
        off = base + k * CHUNK
        pltpu.sync_copy(pos_h.at[pl.ds(off, CHUNK)], idx_p.at[k])
        pltpu.sync_copy(neg_h.at[pl.ds(off, CHUNK)], idx_n.at[k])
    for k in range(NCHUNK):
        for v in range(CHUNK // L):
            sl = pl.ds(v * L, L)
            pid_p[k, sl] = idx_p[k, sl] >> 1
            pid_n[k, sl] = idx_n[k, sl] >> 1

    def fire_wave(k, w, parity):
        v16 = idx_u[pl.ds(k * CHUNK + w * WAVE, L)]
        for e in range(WAVE):
            r = v16[e]
            start = (r >> 7) * 128
            slot = parity * WAVE + e
            pltpu.async_copy(
                umat_t.at[:, pl.ds(start, 128)], ublk.at[slot], sems[parity])

    def drain_wave(parity):
        for e in range(WAVE):
            pltpu.make_async_copy(
                umat_t.at[:, pl.ds(0, 128)],
                ublk.at[parity * WAVE + e], sems[parity]).wait()

    for k in range(NCHUNK):
        cp_p = pltpu.async_copy(imat_h.at[pid_p.at[k]], rows_p, sem_i)
        cp_n = pltpu.async_copy(imat_h.at[pid_n.at[k]], rows_n, sem_i)
        cp_p.wait()
        cp_n.wait()

        def diff_body(g, carry):
            rows16 = g * L + iot
            sl = pl.ds(g * L, L)
            hp = (idx_p[k, sl] & 1) * D
            hn = (idx_n[k, sl] & 1) * D
            for d in range(D):
                iv = plsc.load_gather(rows_p, [rows16, hp + d])
                jv = plsc.load_gather(rows_n, [rows16, hn + d])
                plsc.store_scatter(diff, [rows16, jnp.full((L,), d, jnp.int32)],
                                   iv - jv)
            return carry

        lax.fori_loop(0, GPC, diff_body, 0)

        fire_wave(k, 0, 0)
        fire_wave(k, 1, 1)

        def group_body(g, carry):
            acc = jnp.zeros((L,), jnp.float32)
            for wj in range(WPC // GPC):
                w = g * (WPC // GPC) + wj
                parity = wj % 2
                drain_wave(parity)
                v16 = idx_u[pl.ds(k * CHUNK + w * WAVE, L)]
                for e in range(WAVE):
                    rr = w * WAVE + e
                    r = v16[e]
                    o = r & 127
                    ocol = jnp.full((L,), o, jnp.int32)
                    slot16 = jnp.full((L,), parity * WAVE + e, jnp.int32)
                    sv = jnp.zeros((L,), jnp.float32)
                    for kk in range(D // L):
                        uk = plsc.load_gather(
                            ublk, [slot16, kk * L + iot, ocol])
                        dk = diff[rr, pl.ds(kk * L, L)]
                        sv = sv + uk * dk
                    t = _lane_sum(sv)
                    lane = wj * WAVE + e
                    acc = jnp.where(iot == lane, t, acc)

                @pl.when(w + 2 < WPC)
                def _():
                    fire_wave(k, w + 2, parity)

            out_v[pl.ds(k * CHUNK + g * L, L)] = acc
            return carry

        lax.fori_loop(0, GPC, group_body, 0)

    pltpu.sync_copy(out_v, out_h.at[pl.ds(base, BPW)])


@jax.jit
def _mf(user, pos, neg, umat_t, imat2):
    mesh = plsc.VectorSubcoreMesh(core_axis_name="c", subcore_axis_name="s")
    kfn = functools.partial(
        pl.kernel,
        out_type=jax.ShapeDtypeStruct((BATCH,), jnp.float32),
        mesh=mesh,
        compiler_params=pltpu.CompilerParams(
            needs_layout_passes=False, use_tc_tiling_on_sc=True),
        scratch_types=[
            pltpu.VMEM((BPW + L,), jnp.int32),
            pltpu.VMEM((NCHUNK, CHUNK), jnp.int32),
            pltpu.VMEM((NCHUNK, CHUNK), jnp.int32),
            pltpu.VMEM((NCHUNK, CHUNK), jnp.int32),
            pltpu.VMEM((NCHUNK, CHUNK), jnp.int32),
            pltpu.VMEM((CHUNK, DP), jnp.float32),
            pltpu.VMEM((CHUNK, DP), jnp.float32),
            pltpu.VMEM((CHUNK, D), jnp.float32),
            pltpu.VMEM((NSLOT, D, 128), jnp.float32),
            pltpu.VMEM((BPW,), jnp.float32),
            pltpu.SemaphoreType.DMA,
            pltpu.SemaphoreType.DMA,
            pltpu.SemaphoreType.DMA,
        ],
    )(_mf_body)
    return kfn(user, pos, neg, umat_t, imat2)


def kernel(user, pos, neg, user_mat, item_mat):
    user = user.astype(jnp.int32)
    pos = pos.astype(jnp.int32)
    neg = neg.astype(jnp.int32)
    umat_t = user_mat.T
    imat2 = item_mat.reshape(item_mat.shape[0] // 2, DP)
    return _mf(user, pos, neg, umat_t, imat2)


# speedup vs baseline: 1.3208x; 1.0002x over previous
import functools

import jax
import jax.numpy as jnp
from jax import lax
from jax.experimental import pallas as pl
from jax.experimental.pallas import tpu as pltpu
from jax.experimental.pallas import tpu_sc as plsc

D = 64
DP = 128
BATCH = 16384

NC = 2
NS = 16
L = 16
NW = NC * NS
BPW = BATCH // NW
CHUNK = 128
NCHUNK = BPW // CHUNK
WAVE = 4
NSLOT = 8
WPC = CHUNK // WAVE
GPC = CHUNK // L

def _lane_sum(v):
    iota = lax.iota(jnp.int32, 16)
    for s in (8, 4, 2, 1):
        v = v + jnp.take_along_axis(v, iota ^ s, axis=0)
    return v


def _mf_body(user_h, pos_h, neg_h, umat_t, imat_h, out_h,
             idx_u, idx_p, idx_n, pid_p, pid_n,
             rows_p, rows_n, diff, ublk, out_v,
             sem_i, sem_a, sem_b):
    c = lax.axis_index("c")
    s = lax.axis_index("s")
    wid = s * NC + c
    base = wid * BPW
    sems = (sem_a, sem_b)
    iot = lax.iota(jnp.int32, L)

    pltpu.sync_copy(user_h.at[pl.ds(base, BPW)], idx_u.at[pl.ds(0, BPW)])
    for k in range(NCHUNK):
        off = base + k * CHUNK
        pltpu.sync_copy(pos_h.at[pl.ds(off, CHUNK)], idx_p.at[k])
        pltpu.sync_copy(neg_h.at[pl.ds(off, CHUNK)], idx_n.at[k])
    for k in range(NCHUNK):
        for v in range(CHUNK // L):
            sl = pl.ds(v * L, L)
            pid_p[k, sl] = idx_p[k, sl] >> 1
            pid_n[k, sl] = idx_n[k, sl] >> 1

    def fire_wave(k, w, parity):
        v16 = idx_u[pl.ds(k * CHUNK + w * WAVE, L)]
        for e in range(WAVE):
            r = v16[e]
            start = (r >> 7) * 128
            slot = parity * WAVE + e
            pltpu.async_copy(
                umat_t.at[:, pl.ds(start, 128)], ublk.at[slot], sems[parity])

    def drain_wave(parity):
        for e in range(WAVE):
            pltpu.make_async_copy(
                umat_t.at[:, pl.ds(0, 128)],
                ublk.at[parity * WAVE + e], sems[parity]).wait()

    for k in range(NCHUNK):
        cp_p = pltpu.async_copy(imat_h.at[pid_p.at[k]], rows_p, sem_i)
        cp_n = pltpu.async_copy(imat_h.at[pid_n.at[k]], rows_n, sem_i)
        cp_p.wait()
        cp_n.wait()

        def diff_body(g, carry):
            rows16 = g * L + iot
            sl = pl.ds(g * L, L)
            hp = (idx_p[k, sl] & 1) * D
            hn = (idx_n[k, sl] & 1) * D
            for d in range(D):
                iv = plsc.load_gather(rows_p, [rows16, hp + d])
                jv = plsc.load_gather(rows_n, [rows16, hn + d])
                plsc.store_scatter(diff, [rows16, jnp.full((L,), d, jnp.int32)],
                                   iv - jv)
            return carry

        lax.fori_loop(0, GPC, diff_body, 0)

        fire_wave(k, 0, 0)
        fire_wave(k, 1, 1)

        def group_body(g, carry):
            acc = jnp.zeros((L,), jnp.float32)
            for wj in range(WPC // GPC):
                w = g * (WPC // GPC) + wj
                parity = wj % 2
                drain_wave(parity)
                v16 = idx_u[pl.ds(k * CHUNK + w * WAVE, L)]
                for e in range(WAVE):
                    rr = w * WAVE + e
                    r = v16[e]
                    o = r & 127
                    ocol = jnp.full((L,), o, jnp.int32)
                    slot16 = jnp.full((L,), parity * WAVE + e, jnp.int32)
                    sv = jnp.zeros((L,), jnp.float32)
                    for kk in range(D // L):
                        uk = plsc.load_gather(
                            ublk, [slot16, kk * L + iot, ocol])
                        dk = diff[rr, pl.ds(kk * L, L)]
                        sv = sv + uk * dk
                    t = _lane_sum(sv)
                    lane = wj * WAVE + e
                    acc = jnp.where(iot == lane, t, acc)

                @pl.when(w + 2 < WPC)
                def _():
                    fire_wave(k, w + 2, parity)

            out_v[pl.ds(k * CHUNK + g * L, L)] = acc
            return carry

        lax.fori_loop(0, GPC, group_body, 0)

    pltpu.sync_copy(out_v, out_h.at[pl.ds(base, BPW)])


@jax.jit
def _mf(user, pos, neg, umat_t, imat2):
    mesh = plsc.VectorSubcoreMesh(core_axis_name="c", subcore_axis_name="s")
    kfn = functools.partial(
        pl.kernel,
        out_type=jax.ShapeDtypeStruct((BATCH,), jnp.float32),
        mesh=mesh,
        compiler_params=pltpu.CompilerParams(
            needs_layout_passes=False, use_tc_tiling_on_sc=True,
            skip_device_barrier=True),
        scratch_types=[
            pltpu.VMEM((BPW + L,), jnp.int32),
            pltpu.VMEM((NCHUNK, CHUNK), jnp.int32),
            pltpu.VMEM((NCHUNK, CHUNK), jnp.int32),
            pltpu.VMEM((NCHUNK, CHUNK), jnp.int32),
            pltpu.VMEM((NCHUNK, CHUNK), jnp.int32),
            pltpu.VMEM((CHUNK, DP), jnp.float32),
            pltpu.VMEM((CHUNK, DP), jnp.float32),
            pltpu.VMEM((CHUNK, D), jnp.float32),
            pltpu.VMEM((NSLOT, D, 128), jnp.float32),
            pltpu.VMEM((BPW,), jnp.float32),
            pltpu.SemaphoreType.DMA,
            pltpu.SemaphoreType.DMA,
            pltpu.SemaphoreType.DMA,
        ],
    )(_mf_body)
    return kfn(user, pos, neg, umat_t, imat2)


def kernel(user, pos, neg, user_mat, item_mat):
    user = user.astype(jnp.int32)
    pos = pos.astype(jnp.int32)
    neg = neg.astype(jnp.int32)
    umat_t = user_mat.T
    imat2 = item_mat.reshape(item_mat.shape[0] // 2, DP)
    return _mf(user, pos, neg, umat_t, imat2)
